# baseline (speedup 1.0000x reference)
import functools
import math

import jax
import jax.numpy as jnp
from jax import lax
from jax.experimental import pallas as pl
from jax.experimental.pallas import tpu as pltpu
from jax.experimental.pallas import tpu_sc as plsc

D = 128
AA_V = 24
MOD_V = 15
L_SEQ = 50
BATCH = 4096
COMB = AA_V * MOD_V
ROWS = L_SEQ * BATCH
SCALE = math.sqrt(float(AA_V))

NC = 2
NS = 16
NW = NC * NS
CHUNK = 128
UR = 16


def _prep_body(seq_ref, mods_ref, aa_ref, mod_ref, comb_ref, idx_ref):
    r_a = lax.broadcasted_iota(jnp.int32, (COMB, AA_V), 0)
    c_a = lax.broadcasted_iota(jnp.int32, (COMB, AA_V), 1)
    one_a = (r_a // MOD_V == c_a).astype(jnp.float32)
    r_m = lax.broadcasted_iota(jnp.int32, (COMB, MOD_V), 0)
    c_m = lax.broadcasted_iota(jnp.int32, (COMB, MOD_V), 1)
    one_m = (r_m % MOD_V == c_m).astype(jnp.float32)
    comb_ref[...] = (
        jnp.dot(one_a, aa_ref[...], preferred_element_type=jnp.float32,
                precision=lax.Precision.HIGHEST)
        + jnp.dot(one_m, mod_ref[...], preferred_element_type=jnp.float32,
                  precision=lax.Precision.HIGHEST)) * SCALE
    c = seq_ref[...] * MOD_V + mods_ref[...]
    idx_ref[...] = jnp.transpose(c.T.reshape(L_SEQ, NW, CHUNK), (1, 0, 2))


_prep = pl.pallas_call(
    _prep_body,
    out_shape=(
        jax.ShapeDtypeStruct((COMB, D), jnp.float32),
        jax.ShapeDtypeStruct((NW, L_SEQ, CHUNK), jnp.int32),
    ),
)


@functools.cache
def _sc_gather_fn():
    @functools.partial(
        pl.kernel,
        out_type=jax.ShapeDtypeStruct((ROWS, D), jnp.float32),
        mesh=plsc.VectorSubcoreMesh(core_axis_name="c", subcore_axis_name="s"),
        compiler_params=pltpu.CompilerParams(needs_layout_passes=False),
        scratch_types=[
            pltpu.VMEM((COMB, D), jnp.float32),
            pltpu.VMEM((L_SEQ, D), jnp.float32),
            pltpu.VMEM((L_SEQ, CHUNK), jnp.int32),
            pltpu.VMEM((2, CHUNK, D), jnp.float32),
            pltpu.SemaphoreType.DMA((2,)),
        ],
    )
    def _sc(comb_hbm, pe_hbm, idx_hbm, out_hbm, comb_v, pe_v, idx_v, obuf, wsem):
        wid = lax.axis_index("s") * NC + lax.axis_index("c")
        b0 = wid * CHUNK
        pltpu.sync_copy(comb_hbm, comb_v)
        pltpu.sync_copy(pe_hbm, pe_v)
        pltpu.sync_copy(idx_hbm.at[wid], idx_v)
        lane = lax.broadcasted_iota(jnp.int32, (16,), 0)
        cols = [lane + 16 * j for j in range(8)]

        def do_chunk(l, p):
            pe_regs = [pe_v[l, pl.ds(16 * j, 16)] for j in range(8)]

            def rows(i, carry):
                cvec = idx_v[l, pl.ds(i * UR, UR)]
                for u in range(UR):
                    r = i * UR + u
                    rowv = jnp.broadcast_to(cvec[u], (16,))
                    for j in range(8):
                        v = plsc.load_gather(comb_v, [rowv, cols[j]])
                        obuf[p, r, pl.ds(16 * j, 16)] = v + pe_regs[j]
                return carry

            lax.fori_loop(0, CHUNK // UR, rows, 0)
            pltpu.make_async_copy(
                obuf.at[p], out_hbm.at[pl.ds(l * BATCH + b0, CHUNK)],
                wsem.at[p]).start()

        def wait_write(p):
            pltpu.make_async_copy(
                obuf.at[p], out_hbm.at[pl.ds(b0, CHUNK)], wsem.at[p]).wait()

        def pair(t, carry):
            @pl.when(t > 0)
            def _():
                wait_write(0)

            do_chunk(2 * t, 0)

            @pl.when(t > 0)
            def _():
                wait_write(1)

            do_chunk(2 * t + 1, 1)
            return carry

        lax.fori_loop(0, L_SEQ // 2, pair, 0)
        wait_write(0)
        wait_write(1)

    return _sc


def kernel(seq, mods, aa_table, mod_table, pe):
    seq = seq.astype(jnp.int32)
    mods = mods.astype(jnp.int32)
    pe50 = pe[:L_SEQ, 0, :]
    comb, idxp = _prep(seq, mods, aa_table, mod_table)
    out = _sc_gather_fn()(comb, pe50, idxp)
    return out.reshape(L_SEQ, BATCH, D)


# speedup vs baseline: 1.4393x; 1.4393x over previous
import functools
import math

import jax
import jax.numpy as jnp
from jax import lax
from jax.experimental import pallas as pl
from jax.experimental.pallas import tpu as pltpu
from jax.experimental.pallas import tpu_sc as plsc

D = 128
AA_V = 24
MOD_V = 15
L_SEQ = 50
BATCH = 4096
COMB = AA_V * MOD_V
ROWS = L_SEQ * BATCH
SCALE = math.sqrt(float(AA_V))

NC = 2
NS = 16
NW = NC * NS
CHUNK = 128
UR = 16


def _prep_body(seq_ref, mods_ref, aa_ref, mod_ref, comb_ref, idx_ref):
    r_a = lax.broadcasted_iota(jnp.int32, (COMB, AA_V), 0)
    c_a = lax.broadcasted_iota(jnp.int32, (COMB, AA_V), 1)
    one_a = (r_a // MOD_V == c_a).astype(jnp.float32)
    r_m = lax.broadcasted_iota(jnp.int32, (COMB, MOD_V), 0)
    c_m = lax.broadcasted_iota(jnp.int32, (COMB, MOD_V), 1)
    one_m = (r_m % MOD_V == c_m).astype(jnp.float32)
    comb_ref[...] = (
        jnp.dot(one_a, aa_ref[...], preferred_element_type=jnp.float32,
                precision=lax.Precision.HIGHEST)
        + jnp.dot(one_m, mod_ref[...], preferred_element_type=jnp.float32,
                  precision=lax.Precision.HIGHEST)) * SCALE
    c = (seq_ref[...] * MOD_V + mods_ref[...]) * D
    idx_ref[...] = jnp.transpose(c.T.reshape(L_SEQ, NW, CHUNK), (1, 0, 2))


_prep = pl.pallas_call(
    _prep_body,
    out_shape=(
        jax.ShapeDtypeStruct((COMB, D), jnp.float32),
        jax.ShapeDtypeStruct((NW, L_SEQ, CHUNK), jnp.int32),
    ),
)


@functools.cache
def _sc_gather_fn():
    @functools.partial(
        pl.kernel,
        out_type=jax.ShapeDtypeStruct((ROWS, D), jnp.float32),
        mesh=plsc.VectorSubcoreMesh(core_axis_name="c", subcore_axis_name="s"),
        compiler_params=pltpu.CompilerParams(needs_layout_passes=False),
        scratch_types=[
            pltpu.VMEM((COMB, D), jnp.float32),
            pltpu.VMEM((L_SEQ, D), jnp.float32),
            pltpu.VMEM((L_SEQ, CHUNK), jnp.int32),
            pltpu.VMEM((2, CHUNK, D), jnp.float32),
            pltpu.SemaphoreType.DMA((2,)),
        ],
    )
    def _sc(comb_hbm, pe_hbm, idx_hbm, out_hbm, comb_v, pe_v, idx_v, obuf, wsem):
        wid = lax.axis_index("s") * NC + lax.axis_index("c")
        b0 = wid * CHUNK
        pltpu.sync_copy(comb_hbm, comb_v)
        pltpu.sync_copy(pe_hbm, pe_v)
        pltpu.sync_copy(idx_hbm.at[wid], idx_v)
        lane = lax.broadcasted_iota(jnp.int32, (16,), 0)
        cols = [lane + 16 * j for j in range(8)]

        def do_chunk(l, p):
            pe_regs = [pe_v[l, pl.ds(16 * j, 16)] for j in range(8)]

            @plsc.parallel_loop(0, CHUNK, step=UR, unroll=2)
            def rows(i):
                cvec = idx_v[l, pl.ds(i, UR)]
                for u in range(UR):
                    rowv = jnp.broadcast_to(cvec[u], (16,))
                    for j in range(8):
                        v = plsc.load_gather(comb_v, [rowv + cols[j]])
                        obuf[p, i + u, pl.ds(16 * j, 16)] = v + pe_regs[j]
            pltpu.make_async_copy(
                obuf.at[p], out_hbm.at[pl.ds(l * BATCH + b0, CHUNK)],
                wsem.at[p]).start()

        def wait_write(p):
            pltpu.make_async_copy(
                obuf.at[p], out_hbm.at[pl.ds(b0, CHUNK)], wsem.at[p]).wait()

        def pair(t, carry):
            @pl.when(t > 0)
            def _():
                wait_write(0)

            do_chunk(2 * t, 0)

            @pl.when(t > 0)
            def _():
                wait_write(1)

            do_chunk(2 * t + 1, 1)
            return carry

        lax.fori_loop(0, L_SEQ // 2, pair, 0)
        wait_write(0)
        wait_write(1)

    return _sc


def kernel(seq, mods, aa_table, mod_table, pe):
    seq = seq.astype(jnp.int32)
    mods = mods.astype(jnp.int32)
    pe50 = pe[:L_SEQ, 0, :]
    comb, idxp = _prep(seq, mods, aa_table, mod_table)
    out = _sc_gather_fn()(comb.reshape(COMB * D), pe50, idxp)
    return out.reshape(L_SEQ, BATCH, D)


# speedup vs baseline: 1.9849x; 1.3791x over previous
import functools
import math

import jax
import jax.numpy as jnp
from jax import lax
from jax.experimental import pallas as pl
from jax.experimental.pallas import tpu as pltpu
from jax.experimental.pallas import tpu_sc as plsc

D = 128
AA_V = 24
MOD_V = 15
L_SEQ = 50
BATCH = 4096
COMB = AA_V * MOD_V
ROWS = L_SEQ * BATCH
SCALE = math.sqrt(float(AA_V))

NC = 2
NS = 16
NW = NC * NS
CHUNK = 128
UR = 16
RND = 5
ENG = 3
NROUND = L_SEQ // RND


def _prep_body(seq_ref, mods_ref, aa_ref, mod_ref, pe_ref,
               bt_ref, comb_ref, idx_ref):
    r_a = lax.broadcasted_iota(jnp.int32, (COMB, AA_V), 0)
    c_a = lax.broadcasted_iota(jnp.int32, (COMB, AA_V), 1)
    one_a = (r_a // MOD_V == c_a).astype(jnp.float32)
    r_m = lax.broadcasted_iota(jnp.int32, (COMB, MOD_V), 0)
    c_m = lax.broadcasted_iota(jnp.int32, (COMB, MOD_V), 1)
    one_m = (r_m % MOD_V == c_m).astype(jnp.float32)
    comb = (jnp.dot(one_a, aa_ref[...], preferred_element_type=jnp.float32,
                    precision=lax.Precision.HIGHEST)
            + jnp.dot(one_m, mod_ref[...], preferred_element_type=jnp.float32,
                      precision=lax.Precision.HIGHEST)) * SCALE
    comb_ref[...] = comb
    bt_ref[...] = comb[None, :, :] + pe_ref[...][:, None, :]
    c = seq_ref[...] * MOD_V + mods_ref[...]
    ct = jnp.transpose(c.T.reshape(L_SEQ, NW, CHUNK), (1, 0, 2))
    l_ix = lax.broadcasted_iota(jnp.int32, (NW, L_SEQ, CHUNK), 1)
    is_eng = (l_ix % RND) < ENG
    idx_ref[...] = jnp.where(is_eng, ct + COMB * l_ix, ct * D)


_prep = pl.pallas_call(
    _prep_body,
    out_shape=(
        jax.ShapeDtypeStruct((L_SEQ, COMB, D), jnp.float32),
        jax.ShapeDtypeStruct((COMB, D), jnp.float32),
        jax.ShapeDtypeStruct((NW, L_SEQ, CHUNK), jnp.int32),
    ),
)


@functools.cache
def _sc_gather_fn():
    @functools.partial(
        pl.kernel,
        out_type=jax.ShapeDtypeStruct((ROWS, D), jnp.float32),
        mesh=plsc.VectorSubcoreMesh(core_axis_name="c", subcore_axis_name="s"),
        compiler_params=pltpu.CompilerParams(needs_layout_passes=False),
        scratch_types=[
            pltpu.VMEM((COMB * D,), jnp.float32),
            pltpu.VMEM((L_SEQ, D), jnp.float32),
            pltpu.VMEM((L_SEQ, CHUNK), jnp.int32),
            pltpu.VMEM((2, CHUNK, D), jnp.float32),
            pltpu.VMEM((2, CHUNK, D), jnp.float32),
            pltpu.SemaphoreType.DMA((2,)),
            pltpu.SemaphoreType.DMA((2,)),
            pltpu.SemaphoreType.DMA((2,)),
        ],
    )
    def _sc(bt_hbm, comb_hbm, pe_hbm, idx_hbm, out_hbm,
            comb_v, pe_v, idx_v, ebuf, cbuf, gsem, ewsem, cwsem):
        wid = lax.axis_index("s") * NC + lax.axis_index("c")
        b0 = wid * CHUNK
        pltpu.sync_copy(comb_hbm, comb_v)
        pltpu.sync_copy(pe_hbm, pe_v)
        pltpu.sync_copy(idx_hbm.at[wid], idx_v)
        lane = lax.broadcasted_iota(jnp.int32, (16,), 0)
        cols = [lane + 16 * j for j in range(8)]

        def fire_gather(l, eb):
            pltpu.make_async_copy(
                bt_hbm.at[idx_v.at[l]], ebuf.at[eb], gsem.at[eb]).start()

        def wait_gather(eb):
            pltpu.make_async_copy(
                bt_hbm.at[idx_v.at[0]], ebuf.at[eb], gsem.at[eb]).wait()

        def ewrite(l, eb):
            return pltpu.make_async_copy(
                ebuf.at[eb], out_hbm.at[pl.ds(l * BATCH + b0, CHUNK)],
                ewsem.at[eb])

        def cwrite(l, cb):
            return pltpu.make_async_copy(
                cbuf.at[cb], out_hbm.at[pl.ds(l * BATCH + b0, CHUNK)],
                cwsem.at[cb])

        def compute_chunk(l, cb):
            pe_regs = [pe_v[l, pl.ds(16 * j, 16)] for j in range(8)]

            @plsc.parallel_loop(0, CHUNK, step=UR, unroll=2)
            def rows(i):
                cvec = idx_v[l, pl.ds(i, UR)]
                for u in range(UR):
                    rowv = jnp.broadcast_to(cvec[u], (16,))
                    for j in range(8):
                        v = plsc.load_gather(comb_v, [rowv + cols[j]])
                        cbuf[cb, i + u, pl.ds(16 * j, 16)] = v + pe_regs[j]

        fire_gather(0, 0)
        fire_gather(1, 1)

        def round_(r, carry):
            e0 = r * RND
            wait_gather(0)
            ewrite(e0, 0).start()
            wait_gather(1)
            ewrite(e0 + 1, 1).start()
            ewrite(0, 0).wait()
            fire_gather(e0 + 2, 0)
            @pl.when(r > 0)
            def _():
                cwrite(0, 0).wait()

            compute_chunk(e0 + 3, 0)
            cwrite(e0 + 3, 0).start()
            wait_gather(0)
            ewrite(e0 + 2, 0).start()
            ewrite(0, 1).wait()

            @pl.when(r < NROUND - 1)
            def _():
                fire_gather(e0 + RND + 1, 1)

            @pl.when(r > 0)
            def _():
                cwrite(0, 1).wait()

            compute_chunk(e0 + 4, 1)
            cwrite(e0 + 4, 1).start()
            ewrite(0, 0).wait()

            @pl.when(r < NROUND - 1)
            def _():
                fire_gather(e0 + RND, 0)

            return carry

        lax.fori_loop(0, NROUND, round_, 0)
        cwrite(0, 0).wait()
        cwrite(0, 1).wait()

    return _sc


def kernel(seq, mods, aa_table, mod_table, pe):
    seq = seq.astype(jnp.int32)
    mods = mods.astype(jnp.int32)
    pe50 = pe[:L_SEQ, 0, :]
    bt, comb, idxp = _prep(seq, mods, aa_table, mod_table, pe50)
    out = _sc_gather_fn()(
        bt.reshape(L_SEQ * COMB, D), comb.reshape(COMB * D), pe50, idxp)
    return out.reshape(L_SEQ, BATCH, D)


# speedup vs baseline: 2.4193x; 1.2189x over previous
import functools
import math

import jax
import jax.numpy as jnp
from jax import lax
from jax.experimental import pallas as pl
from jax.experimental.pallas import tpu as pltpu
from jax.experimental.pallas import tpu_sc as plsc

D = 128
AA_V = 24
MOD_V = 15
L_SEQ = 50
BATCH = 4096
COMB = AA_V * MOD_V
ROWS = L_SEQ * BATCH
SCALE = math.sqrt(float(AA_V))

NC = 2
NS = 16
NW = NC * NS
ROWS_PER_W = ROWS // NW
CHUNK = 128
NCHUNK = ROWS_PER_W // CHUNK
NBUF = 5
NROUND = NCHUNK // NBUF


def _prep_body(seq_ref, mods_ref, aa_ref, mod_ref, pe_ref, bt_ref, idx_ref):
    r_a = lax.broadcasted_iota(jnp.int32, (COMB, AA_V), 0)
    c_a = lax.broadcasted_iota(jnp.int32, (COMB, AA_V), 1)
    one_a = (r_a // MOD_V == c_a).astype(jnp.float32)
    r_m = lax.broadcasted_iota(jnp.int32, (COMB, MOD_V), 0)
    c_m = lax.broadcasted_iota(jnp.int32, (COMB, MOD_V), 1)
    one_m = (r_m % MOD_V == c_m).astype(jnp.float32)
    comb = (jnp.dot(one_a, aa_ref[...], preferred_element_type=jnp.float32,
                    precision=lax.Precision.HIGHEST)
            + jnp.dot(one_m, mod_ref[...], preferred_element_type=jnp.float32,
                      precision=lax.Precision.HIGHEST))
    bt_ref[...] = comb[None, :, :] * SCALE + pe_ref[...][:, None, :]
    c = seq_ref[...] * MOD_V + mods_ref[...]
    idx_ref[...] = c.T + COMB * lax.broadcasted_iota(jnp.int32, (L_SEQ, BATCH), 0)


_prep = pl.pallas_call(
    _prep_body,
    out_shape=(
        jax.ShapeDtypeStruct((L_SEQ, COMB, D), jnp.float32),
        jax.ShapeDtypeStruct((L_SEQ, BATCH), jnp.int32),
    ),
)


@functools.cache
def _sc_gather_fn():
    @functools.partial(
        pl.kernel,
        out_type=jax.ShapeDtypeStruct((ROWS, D), jnp.float32),
        mesh=plsc.VectorSubcoreMesh(core_axis_name="c", subcore_axis_name="s"),
        scratch_types=[
            pltpu.VMEM((NCHUNK, CHUNK), jnp.int32),
            pltpu.VMEM((NBUF, CHUNK, D), jnp.float32),
            pltpu.SemaphoreType.DMA((NBUF,)),
            pltpu.SemaphoreType.DMA((NBUF,)),
        ],
    )
    def _sc_gather(bt_hbm, idx_hbm, out_hbm, idx_v, rows_v, gsem, wsem):
        wid = lax.axis_index("s") * NC + lax.axis_index("c")
        base = wid * ROWS_PER_W
        pltpu.sync_copy(idx_hbm.at[wid], idx_v)

        def gather(k, j):
            pltpu.make_async_copy(
                bt_hbm.at[idx_v.at[k]], rows_v.at[j], gsem.at[j]).start()

        def write(k, j):
            return pltpu.make_async_copy(
                rows_v.at[j], out_hbm.at[pl.ds(base + k * CHUNK, CHUNK)],
                wsem.at[j])

        for j in range(NBUF):
            gather(j, j)

        def round_(p, carry):
            for j in range(NBUF):
                k = p * NBUF + j
                pltpu.make_async_copy(
                    bt_hbm.at[idx_v.at[k]], rows_v.at[j], gsem.at[j]).wait()
                write(k, j).start()
            for j in range(NBUF):
                write(p * NBUF + j, j).wait()

                @pl.when(p < NROUND - 1)
                def _():
                    gather((p + 1) * NBUF + j, j)
            return carry

        lax.fori_loop(0, NROUND, round_, 0)

    return _sc_gather


def kernel(seq, mods, aa_table, mod_table, pe):
    seq = seq.astype(jnp.int32)
    mods = mods.astype(jnp.int32)
    pe50 = pe[:L_SEQ, 0, :]
    bt, idx = _prep(seq, mods, aa_table, mod_table, pe50)
    out = _sc_gather_fn()(bt.reshape(L_SEQ * COMB, D), idx.reshape(NW, NCHUNK, CHUNK))
    return out.reshape(L_SEQ, BATCH, D)
